# baseline (speedup 1.0000x reference)
import jax
import jax.numpy as jnp
from jax import lax
from jax.experimental import pallas as pl
from jax.experimental.pallas import tpu as pltpu
from jax.experimental.pallas import tpu_sc as plsc
import numpy as np

CLS_ID = 101
SEP_ID = 102
MAX_LENGTH = 4096
B = 16
LQ = 64
HALF = MAX_LENGTH // 2
NCHUNK = 2 * B
NPRE = 8
NVREG = HALF // 16


def _body(q_hbm, d_hbm, lens_hbm,
          tok_hbm, mask_hbm, pid_hbm,
          q_v, d_v, lens_v, tok_v, mask_v, pid_v, sem_in, sem_out):
    nc = 2
    wid = lax.axis_index("s") * nc + lax.axis_index("c")
    row = wid // 2
    half = wid % 2
    base = (half * HALF).astype(jnp.int32)

    c_lens = pltpu.async_copy(lens_hbm, lens_v, sem_in)
    c_d = pltpu.async_copy(d_hbm.at[row], d_v, sem_in)
    c_q = pltpu.async_copy(q_hbm.at[row], q_v, sem_in)
    c_lens.wait()

    row_v = jnp.full((16,), row, jnp.int32)
    qlen = plsc.load_gather(lens_v, [row_v])
    dlen = plsc.load_gather(lens_v, [row_v + np.int32(B)])
    dlen_eff = jnp.minimum(dlen, np.int32(MAX_LENGTH - 2) - qlen)
    qoff = qlen + np.int32(2)
    qlen1 = qlen + np.int32(1)
    total = qoff + dlen_eff
    lane = lax.iota(jnp.int32, 16)

    @plsc.parallel_loop(jnp.int32(0), jnp.int32(NVREG),
                        step=jnp.int32(1), unroll=8)
    def _mp(i):
        off = i * np.int32(16)
        p = lane + base + off
        in_seq = p < total
        mask_v[pl.ds(off, 16)] = jnp.where(in_seq, np.float32(1.0),
                                           np.float32(0.0))
        pid_v[pl.ds(off, 16)] = jnp.where(p <= qlen, p,
                                jnp.where(in_seq, p - qlen1, np.int32(0)))

    o_mask = pltpu.async_copy(mask_v, mask_hbm.at[wid], sem_out)
    o_pid = pltpu.async_copy(pid_v, pid_hbm.at[wid], sem_out)

    c_q.wait()
    c_d.wait()

    for i in range(NPRE):
        p = lane + base + np.int32(16 * i)
        q_idx = jnp.clip(p - np.int32(1), np.int32(0), np.int32(LQ - 1))
        q_tok = plsc.load_gather(q_v, [q_idx])
        d_idx = jnp.maximum(p - qoff, np.int32(0))
        d_tok = plsc.load_gather(d_v, [d_idx])
        tok_v[pl.ds(16 * i, 16)] = (
            jnp.where(p == np.int32(0), np.int32(CLS_ID),
            jnp.where(p <= qlen, q_tok,
            jnp.where(p == qlen1, np.int32(SEP_ID),
            jnp.where(p < total, d_tok, np.int32(0))))))

    @plsc.parallel_loop(jnp.int32(NPRE), jnp.int32(NVREG),
                        step=jnp.int32(1), unroll=8)
    def _bulk(i):
        off = i * np.int32(16)
        p = lane + base + off
        d_tok = plsc.load_gather(d_v, [p - qoff])
        tok_v[pl.ds(off, 16)] = jnp.where(p < total, d_tok, np.int32(0))

    o_tok = pltpu.async_copy(tok_v, tok_hbm.at[wid], sem_out)
    o_mask.wait()
    o_pid.wait()
    o_tok.wait()


_MESH = plsc.VectorSubcoreMesh(core_axis_name="c", subcore_axis_name="s")

_run = pl.kernel(
    _body,
    out_type=(
        jax.ShapeDtypeStruct((NCHUNK, HALF), jnp.int32),
        jax.ShapeDtypeStruct((NCHUNK, HALF), jnp.float32),
        jax.ShapeDtypeStruct((NCHUNK, HALF), jnp.int32),
    ),
    mesh=_MESH,
    compiler_params=pltpu.CompilerParams(needs_layout_passes=False),
    scratch_types=[
        pltpu.VMEM((LQ,), jnp.int32),
        pltpu.VMEM((MAX_LENGTH,), jnp.int32),
        pltpu.VMEM((2 * B,), jnp.int32),
        pltpu.VMEM((HALF,), jnp.int32),
        pltpu.VMEM((HALF,), jnp.float32),
        pltpu.VMEM((HALF,), jnp.int32),
        pltpu.SemaphoreType.DMA,
        pltpu.SemaphoreType.DMA,
    ],
)


def kernel(query, document, q_lens, d_lens):
    q32 = query.astype(jnp.int32)
    d32 = document.astype(jnp.int32)
    lens = jnp.concatenate([q_lens, d_lens])
    tok, mask, pid = _run(q32, d32, lens)
    tok = tok.reshape(B, MAX_LENGTH).astype(query.dtype)
    mask = mask.reshape(B, MAX_LENGTH)
    pid = pid.reshape(B, MAX_LENGTH).astype(jnp.int64)
    return tok, mask, pid


# speedup vs baseline: 1.0640x; 1.0640x over previous
import jax
import jax.numpy as jnp
from jax import lax
from jax.experimental import pallas as pl
from jax.experimental.pallas import tpu as pltpu
from jax.experimental.pallas import tpu_sc as plsc
import numpy as np

CLS_ID = 101
SEP_ID = 102
MAX_LENGTH = 4096
B = 16
LQ = 64
HALF = MAX_LENGTH // 2
NCHUNK = 2 * B
NPRE = 8
NVREG = HALF // 16
NMID = NVREG // 2


def _body(q_hbm, d_hbm, ql_hbm, dl_hbm,
          tok_hbm, mask_hbm, pid_hbm,
          q_v, d_v, lens_v, tok_v, mask_v, pid_v, sem_in, sem_out):
    nc = 2
    wid = lax.axis_index("s") * nc + lax.axis_index("c")
    row = wid // 2
    half = wid % 2
    base = (half * HALF).astype(jnp.int32)

    c_ql = pltpu.async_copy(ql_hbm, lens_v.at[pl.ds(0, B)], sem_in)
    c_dl = pltpu.async_copy(dl_hbm, lens_v.at[pl.ds(B, B)], sem_in)
    c_d1 = pltpu.async_copy(d_hbm.at[row, pl.ds(0, 128)],
                            d_v.at[pl.ds(0, 128)], sem_in)
    c_q = pltpu.async_copy(q_hbm.at[row], q_v, sem_in)
    c_d2 = pltpu.async_copy(d_hbm.at[row, pl.ds(128, MAX_LENGTH - 128)],
                            d_v.at[pl.ds(128, MAX_LENGTH - 128)], sem_in)
    c_ql.wait()
    c_dl.wait()

    row_v = jnp.full((16,), row, jnp.int32)
    qlen = plsc.load_gather(lens_v, [row_v])
    dlen = plsc.load_gather(lens_v, [row_v + np.int32(B)])
    dlen_eff = jnp.minimum(dlen, np.int32(MAX_LENGTH - 2) - qlen)
    qoff = qlen + np.int32(2)
    qlen1 = qlen + np.int32(1)
    total = qoff + dlen_eff
    lane = lax.iota(jnp.int32, 16)

    c_q.wait()
    c_d1.wait()

    @pl.when(half == 0)
    def _prefix():
        for i in range(NPRE):
            p = lane + np.int32(16 * i)
            q_idx = jnp.clip(p - np.int32(1), np.int32(0), np.int32(LQ - 1))
            q_tok = plsc.load_gather(q_v, [q_idx])
            d_idx = jnp.clip(p - qoff, np.int32(0), np.int32(126))
            d_tok = plsc.load_gather(d_v, [d_idx])
            tok_v[pl.ds(16 * i, 16)] = (
                jnp.where(p == np.int32(0), np.int32(CLS_ID),
                jnp.where(p <= qlen, q_tok,
                jnp.where(p == qlen1, np.int32(SEP_ID),
                jnp.where(p < total, d_tok, np.int32(0))))))

    @plsc.parallel_loop(jnp.int32(0), jnp.int32(NVREG),
                        step=jnp.int32(1), unroll=8)
    def _mp(i):
        off = i * np.int32(16)
        p = lane + base + off
        in_seq = p < total
        mask_v[pl.ds(off, 16)] = jnp.where(in_seq, np.float32(1.0),
                                           np.float32(0.0))
        pid_v[pl.ds(off, 16)] = jnp.where(p <= qlen, p,
                                jnp.where(in_seq, p - qlen1, np.int32(0)))

    o_mask = pltpu.async_copy(mask_v, mask_hbm.at[wid], sem_out)
    o_pid = pltpu.async_copy(pid_v, pid_hbm.at[wid], sem_out)

    c_d2.wait()

    bulk_lo = jnp.where(half == 0, jnp.int32(NPRE), jnp.int32(0))

    @plsc.parallel_loop(bulk_lo, jnp.int32(NMID),
                        step=jnp.int32(1), unroll=8)
    def _bulk_a(i):
        off = i * np.int32(16)
        p = lane + base + off
        d_tok = plsc.load_gather(d_v, [p - qoff])
        tok_v[pl.ds(off, 16)] = jnp.where(p < total, d_tok, np.int32(0))

    o_tok1 = pltpu.async_copy(tok_v.at[pl.ds(0, NMID * 16)],
                              tok_hbm.at[wid, pl.ds(0, NMID * 16)], sem_out)

    @plsc.parallel_loop(jnp.int32(NMID), jnp.int32(NVREG),
                        step=jnp.int32(1), unroll=8)
    def _bulk_b(i):
        off = i * np.int32(16)
        p = lane + base + off
        d_tok = plsc.load_gather(d_v, [p - qoff])
        tok_v[pl.ds(off, 16)] = jnp.where(p < total, d_tok, np.int32(0))

    o_tok2 = pltpu.async_copy(tok_v.at[pl.ds(NMID * 16, NMID * 16)],
                              tok_hbm.at[wid, pl.ds(NMID * 16, NMID * 16)],
                              sem_out)
    o_mask.wait()
    o_pid.wait()
    o_tok1.wait()
    o_tok2.wait()


_MESH = plsc.VectorSubcoreMesh(core_axis_name="c", subcore_axis_name="s")

_run = pl.kernel(
    _body,
    out_type=(
        jax.ShapeDtypeStruct((NCHUNK, HALF), jnp.int32),
        jax.ShapeDtypeStruct((NCHUNK, HALF), jnp.float32),
        jax.ShapeDtypeStruct((NCHUNK, HALF), jnp.int32),
    ),
    mesh=_MESH,
    compiler_params=pltpu.CompilerParams(needs_layout_passes=False),
    scratch_types=[
        pltpu.VMEM((LQ,), jnp.int32),
        pltpu.VMEM((MAX_LENGTH,), jnp.int32),
        pltpu.VMEM((2 * B,), jnp.int32),
        pltpu.VMEM((HALF,), jnp.int32),
        pltpu.VMEM((HALF,), jnp.float32),
        pltpu.VMEM((HALF,), jnp.int32),
        pltpu.SemaphoreType.DMA,
        pltpu.SemaphoreType.DMA,
    ],
)


def kernel(query, document, q_lens, d_lens):
    q32 = query.astype(jnp.int32)
    d32 = document.astype(jnp.int32)
    tok, mask, pid = _run(q32, d32, q_lens, d_lens)
    tok = tok.reshape(B, MAX_LENGTH).astype(query.dtype)
    mask = mask.reshape(B, MAX_LENGTH)
    pid = pid.reshape(B, MAX_LENGTH).astype(jnp.int64)
    return tok, mask, pid


# speedup vs baseline: 1.0654x; 1.0013x over previous
import jax
import jax.numpy as jnp
from jax import lax
from jax.experimental import pallas as pl
from jax.experimental.pallas import tpu as pltpu
from jax.experimental.pallas import tpu_sc as plsc
import numpy as np

CLS_ID = 101
SEP_ID = 102
MAX_LENGTH = 4096
B = 16
LQ = 64
HALF = MAX_LENGTH // 2
NCHUNK = 2 * B
NPRE = 8
NVREG = HALF // 16
NMID = NVREG // 2


def _body(q_hbm, d_hbm, ql_hbm, dl_hbm,
          tok_hbm, mask_hbm, pid_hbm,
          q_v, d_v, lens_v, tok_v, mask_v, pid_v, sem_in, sem_out):
    nc = 2
    wid = lax.axis_index("s") * nc + lax.axis_index("c")
    row = wid // 2
    half = wid % 2
    base = (half * HALF).astype(jnp.int32)

    c_ql = pltpu.async_copy(ql_hbm, lens_v.at[pl.ds(0, B)], sem_in)
    c_dl = pltpu.async_copy(dl_hbm, lens_v.at[pl.ds(B, B)], sem_in)
    c_d1 = pltpu.async_copy(d_hbm.at[row, pl.ds(0, 128)],
                            d_v.at[pl.ds(0, 128)], sem_in)
    c_q = pltpu.async_copy(q_hbm.at[row], q_v, sem_in)
    c_d2 = pltpu.async_copy(d_hbm.at[row, pl.ds(128, MAX_LENGTH - 128)],
                            d_v.at[pl.ds(128, MAX_LENGTH - 128)], sem_in)
    c_ql.wait()
    c_dl.wait()

    row_v = jnp.full((16,), row, jnp.int32)
    qlen = plsc.load_gather(lens_v, [row_v])
    dlen = plsc.load_gather(lens_v, [row_v + np.int32(B)])
    dlen_eff = jnp.minimum(dlen, np.int32(MAX_LENGTH - 2) - qlen)
    qoff = qlen + np.int32(2)
    qlen1 = qlen + np.int32(1)
    total = qoff + dlen_eff
    lane = lax.iota(jnp.int32, 16)

    @plsc.parallel_loop(jnp.int32(0), jnp.int32(NVREG),
                        step=jnp.int32(1), unroll=8)
    def _mp(i):
        off = i * np.int32(16)
        p = lane + base + off
        in_seq = p < total
        mask_v[pl.ds(off, 16)] = jnp.where(in_seq, np.float32(1.0),
                                           np.float32(0.0))
        pid_v[pl.ds(off, 16)] = jnp.where(p <= qlen, p,
                                jnp.where(in_seq, p - qlen1, np.int32(0)))

    o_mask = pltpu.async_copy(mask_v, mask_hbm.at[wid], sem_out)
    o_pid = pltpu.async_copy(pid_v, pid_hbm.at[wid], sem_out)

    c_q.wait()
    c_d1.wait()

    @pl.when(half == 0)
    def _prefix():
        for i in range(NPRE):
            p = lane + np.int32(16 * i)
            q_idx = jnp.clip(p - np.int32(1), np.int32(0), np.int32(LQ - 1))
            q_tok = plsc.load_gather(q_v, [q_idx])
            d_idx = jnp.clip(p - qoff, np.int32(0), np.int32(126))
            d_tok = plsc.load_gather(d_v, [d_idx])
            tok_v[pl.ds(16 * i, 16)] = (
                jnp.where(p == np.int32(0), np.int32(CLS_ID),
                jnp.where(p <= qlen, q_tok,
                jnp.where(p == qlen1, np.int32(SEP_ID),
                jnp.where(p < total, d_tok, np.int32(0))))))

    c_d2.wait()

    bulk_lo = jnp.where(half == 0, jnp.int32(NPRE), jnp.int32(0))

    @plsc.parallel_loop(bulk_lo, jnp.int32(NMID),
                        step=jnp.int32(1), unroll=8)
    def _bulk_a(i):
        off = i * np.int32(16)
        p = lane + base + off
        d_tok = plsc.load_gather(d_v, [p - qoff])
        tok_v[pl.ds(off, 16)] = jnp.where(p < total, d_tok, np.int32(0))

    o_tok1 = pltpu.async_copy(tok_v.at[pl.ds(0, NMID * 16)],
                              tok_hbm.at[wid, pl.ds(0, NMID * 16)], sem_out)

    @plsc.parallel_loop(jnp.int32(NMID), jnp.int32(96),
                        step=jnp.int32(1), unroll=8)
    def _bulk_b(i):
        off = i * np.int32(16)
        p = lane + base + off
        d_tok = plsc.load_gather(d_v, [p - qoff])
        tok_v[pl.ds(off, 16)] = jnp.where(p < total, d_tok, np.int32(0))

    o_tok2 = pltpu.async_copy(tok_v.at[pl.ds(NMID * 16, 512)],
                              tok_hbm.at[wid, pl.ds(NMID * 16, 512)],
                              sem_out)

    @plsc.parallel_loop(jnp.int32(96), jnp.int32(NVREG),
                        step=jnp.int32(1), unroll=8)
    def _bulk_c(i):
        off = i * np.int32(16)
        p = lane + base + off
        d_tok = plsc.load_gather(d_v, [p - qoff])
        tok_v[pl.ds(off, 16)] = jnp.where(p < total, d_tok, np.int32(0))

    o_tok3 = pltpu.async_copy(tok_v.at[pl.ds(96 * 16, 512)],
                              tok_hbm.at[wid, pl.ds(96 * 16, 512)],
                              sem_out)
    o_mask.wait()
    o_pid.wait()
    o_tok1.wait()
    o_tok2.wait()
    o_tok3.wait()


_MESH = plsc.VectorSubcoreMesh(core_axis_name="c", subcore_axis_name="s")

_run = pl.kernel(
    _body,
    out_type=(
        jax.ShapeDtypeStruct((NCHUNK, HALF), jnp.int32),
        jax.ShapeDtypeStruct((NCHUNK, HALF), jnp.float32),
        jax.ShapeDtypeStruct((NCHUNK, HALF), jnp.int32),
    ),
    mesh=_MESH,
    compiler_params=pltpu.CompilerParams(needs_layout_passes=False),
    scratch_types=[
        pltpu.VMEM((LQ,), jnp.int32),
        pltpu.VMEM((MAX_LENGTH,), jnp.int32),
        pltpu.VMEM((2 * B,), jnp.int32),
        pltpu.VMEM((HALF,), jnp.int32),
        pltpu.VMEM((HALF,), jnp.float32),
        pltpu.VMEM((HALF,), jnp.int32),
        pltpu.SemaphoreType.DMA,
        pltpu.SemaphoreType.DMA,
    ],
)


def kernel(query, document, q_lens, d_lens):
    q32 = query.astype(jnp.int32)
    d32 = document.astype(jnp.int32)
    tok, mask, pid = _run(q32, d32, q_lens, d_lens)
    tok = tok.reshape(B, MAX_LENGTH).astype(query.dtype)
    mask = mask.reshape(B, MAX_LENGTH)
    pid = pid.reshape(B, MAX_LENGTH).astype(jnp.int64)
    return tok, mask, pid
